# baseline (speedup 1.0000x reference)
import functools

import jax
import jax.numpy as jnp
from jax import lax
from jax.experimental import pallas as pl
from jax.experimental.pallas import tpu as pltpu
from jax.experimental.pallas import tpu_sc as plsc

BATCH = 16384
HIST = 200
EMBED = 2
PADW = 8
N_VOCAB = 1000000
N_TOTAL = BATCH * HIST
NC, NS = 2, 16
NW = NC * NS
PER_W = N_TOTAL // NW
CHUNK = 6400
N_CHUNKS = PER_W // CHUNK

_mesh = plsc.VectorSubcoreMesh(core_axis_name="c", subcore_axis_name="s")


@functools.partial(
    pl.kernel,
    out_type=jax.ShapeDtypeStruct((N_TOTAL, PADW), jnp.float32),
    mesh=_mesh,
    scratch_types=[
        pltpu.VMEM((CHUNK,), jnp.int32),
        pltpu.VMEM((CHUNK, PADW), jnp.float32),
        pltpu.SemaphoreType.DMA,
    ],
    compiler_params=pltpu.CompilerParams(use_tc_tiling_on_sc=False),
)
def _gather_kernel(idx_hbm, w_hbm, rows_hbm, idx_v, rows_v, sem):
    wid = lax.axis_index("s") * NC + lax.axis_index("c")
    base = wid * PER_W
    for c in range(N_CHUNKS):
        off = base + c * CHUNK
        pltpu.sync_copy(idx_hbm.at[pl.ds(off, CHUNK)], idx_v)
        pltpu.async_copy(w_hbm.at[idx_v], rows_v, sem).wait()
        pltpu.sync_copy(rows_v, rows_hbm.at[pl.ds(off, CHUNK)])


def kernel(x, W):
    xf = x.reshape(N_TOTAL).astype(jnp.int32)
    w0, w1 = lax.optimization_barrier((W[:, 0], W[:, 1]))
    lane = lax.broadcasted_iota(jnp.int32, (N_VOCAB // 64, 512), 1) % PADW
    aa = jnp.repeat(w0.reshape(N_VOCAB // 64, 64), PADW, axis=1)
    bb = jnp.repeat(w1.reshape(N_VOCAB // 64, 64), PADW, axis=1)
    wpad = jnp.where(lane == 0, aa, jnp.where(lane == 1, bb, 0.0))
    w_rm = wpad.reshape(N_VOCAB, PADW)
    rows = _gather_kernel(xf, w_rm)
    o0 = rows[:, 0].reshape(BATCH, HIST)
    o1 = rows[:, 1].reshape(BATCH, HIST)
    return jnp.stack([o0, o1], axis=-1)


# speedup vs baseline: 9.1241x; 9.1241x over previous
import functools

import jax
import jax.numpy as jnp
from jax import lax
from jax.experimental import pallas as pl
from jax.experimental.pallas import tpu as pltpu
from jax.experimental.pallas import tpu_sc as plsc

BATCH = 16384
HIST = 200
EMBED = 2
N_TOTAL = BATCH * HIST
NC, NS = 2, 16
NW = NC * NS
PER_W = N_TOTAL // NW
CHUNK = 10240
N_CHUNKS = PER_W // CHUNK

_mesh = plsc.VectorSubcoreMesh(core_axis_name="c", subcore_axis_name="s")


@functools.partial(
    pl.kernel,
    out_type=(
        jax.ShapeDtypeStruct((N_TOTAL,), jnp.float32),
        jax.ShapeDtypeStruct((N_TOTAL,), jnp.float32),
    ),
    mesh=_mesh,
    scratch_types=[
        pltpu.VMEM((2, CHUNK), jnp.int32),
        pltpu.VMEM((2, CHUNK), jnp.float32),
        pltpu.VMEM((2, CHUNK), jnp.float32),
        pltpu.SemaphoreType.DMA,
        pltpu.SemaphoreType.DMA,
        pltpu.SemaphoreType.DMA,
        pltpu.SemaphoreType.DMA,
    ],
    compiler_params=pltpu.CompilerParams(use_tc_tiling_on_sc=False),
)
def _gather_kernel(idx_hbm, w0_hbm, w1_hbm, o0_hbm, o1_hbm,
                   idx_v, g0_v, g1_v, sem_ga, sem_gb, sem_wa, sem_wb):
    wid = lax.axis_index("s") * NC + lax.axis_index("c")
    base = wid * PER_W
    sem_g = (sem_ga, sem_gb)
    sem_w = (sem_wa, sem_wb)

    def start_gathers(c, p):
        cp0 = pltpu.async_copy(w0_hbm.at[idx_v.at[p]], g0_v.at[p], sem_g[p])
        cp1 = pltpu.async_copy(w1_hbm.at[idx_v.at[p]], g1_v.at[p], sem_g[p])
        return (cp0, cp1)

    pltpu.sync_copy(idx_hbm.at[pl.ds(base, CHUNK)], idx_v.at[0])
    g = start_gathers(0, 0)
    w_prev = [None, None]

    for c in range(N_CHUNKS):
        p = c % 2
        q = 1 - p
        if c + 1 < N_CHUNKS:
            off_n = base + (c + 1) * CHUNK
            pltpu.sync_copy(idx_hbm.at[pl.ds(off_n, CHUNK)], idx_v.at[q])
        if w_prev[q] is not None:
            w_prev[q][0].wait()
            w_prev[q][1].wait()
            w_prev[q] = None
        g[0].wait()
        g[1].wait()
        if c + 1 < N_CHUNKS:
            g = start_gathers(c + 1, q)
        off = base + c * CHUNK
        cw0 = pltpu.async_copy(g0_v.at[p], o0_hbm.at[pl.ds(off, CHUNK)],
                               sem_w[p])
        cw1 = pltpu.async_copy(g1_v.at[p], o1_hbm.at[pl.ds(off, CHUNK)],
                               sem_w[p])
        w_prev[p] = (cw0, cw1)

    for p in range(2):
        if w_prev[p] is not None:
            w_prev[p][0].wait()
            w_prev[p][1].wait()


def kernel(x, W):
    xf = x.reshape(N_TOTAL).astype(jnp.int32)
    w0 = W[:, 0]
    w1 = W[:, 1]
    o0, o1 = _gather_kernel(xf, w0, w1)
    return jnp.stack(
        [o0.reshape(BATCH, HIST), o1.reshape(BATCH, HIST)], axis=-1)


# speedup vs baseline: 9.1464x; 1.0024x over previous
import functools

import jax
import jax.numpy as jnp
from jax import lax
from jax.experimental import pallas as pl
from jax.experimental.pallas import tpu as pltpu
from jax.experimental.pallas import tpu_sc as plsc

BATCH = 16384
HIST = 200
EMBED = 2
N_TOTAL = BATCH * HIST
NC, NS = 2, 16
NW = NC * NS
PER_W = N_TOTAL // NW
CHUNK = 20480
N_CHUNKS = PER_W // CHUNK

_mesh = plsc.VectorSubcoreMesh(core_axis_name="c", subcore_axis_name="s")


@functools.partial(
    pl.kernel,
    out_type=(
        jax.ShapeDtypeStruct((N_TOTAL,), jnp.float32),
        jax.ShapeDtypeStruct((N_TOTAL,), jnp.float32),
    ),
    mesh=_mesh,
    scratch_types=[
        pltpu.VMEM((2, CHUNK), jnp.int32),
        pltpu.VMEM((2, CHUNK), jnp.float32),
        pltpu.VMEM((2, CHUNK), jnp.float32),
        pltpu.SemaphoreType.DMA,
        pltpu.SemaphoreType.DMA,
        pltpu.SemaphoreType.DMA,
        pltpu.SemaphoreType.DMA,
    ],
    compiler_params=pltpu.CompilerParams(use_tc_tiling_on_sc=False),
)
def _gather_kernel(idx_hbm, w0_hbm, w1_hbm, o0_hbm, o1_hbm,
                   idx_v, g0_v, g1_v, sem_ga, sem_gb, sem_wa, sem_wb):
    wid = lax.axis_index("s") * NC + lax.axis_index("c")
    base = wid * PER_W
    sem_g = (sem_ga, sem_gb)
    sem_w = (sem_wa, sem_wb)

    def start_gathers(c, p):
        cp0 = pltpu.async_copy(w0_hbm.at[idx_v.at[p]], g0_v.at[p], sem_g[p])
        cp1 = pltpu.async_copy(w1_hbm.at[idx_v.at[p]], g1_v.at[p], sem_g[p])
        return (cp0, cp1)

    pltpu.sync_copy(idx_hbm.at[pl.ds(base, CHUNK)], idx_v.at[0])
    g = start_gathers(0, 0)
    w_prev = [None, None]

    for c in range(N_CHUNKS):
        p = c % 2
        q = 1 - p
        if c + 1 < N_CHUNKS:
            off_n = base + (c + 1) * CHUNK
            pltpu.sync_copy(idx_hbm.at[pl.ds(off_n, CHUNK)], idx_v.at[q])
        if w_prev[q] is not None:
            w_prev[q][0].wait()
            w_prev[q][1].wait()
            w_prev[q] = None
        g[0].wait()
        g[1].wait()
        if c + 1 < N_CHUNKS:
            g = start_gathers(c + 1, q)
        off = base + c * CHUNK
        cw0 = pltpu.async_copy(g0_v.at[p], o0_hbm.at[pl.ds(off, CHUNK)],
                               sem_w[p])
        cw1 = pltpu.async_copy(g1_v.at[p], o1_hbm.at[pl.ds(off, CHUNK)],
                               sem_w[p])
        w_prev[p] = (cw0, cw1)

    for p in range(2):
        if w_prev[p] is not None:
            w_prev[p][0].wait()
            w_prev[p][1].wait()


def kernel(x, W):
    xf = x.reshape(N_TOTAL).astype(jnp.int32)
    w0 = W[:, 0]
    w1 = W[:, 1]
    o0, o1 = _gather_kernel(xf, w0, w1)
    return jnp.stack(
        [o0.reshape(BATCH, HIST), o1.reshape(BATCH, HIST)], axis=-1)
